# Initial kernel scaffold; baseline (speedup 1.0000x reference)
import jax
import jax.numpy as jnp
from jax.experimental import pallas as pl


def kernel(x, edge_index, edge_attr, W1, b1, gamma1, beta1, a1):
    raise NotImplementedError("write your pallas kernel here")


# speedup vs baseline: 10.7907x; 10.7907x over previous
import functools

import jax
import jax.numpy as jnp
from jax import lax
from jax.experimental import pallas as pl
from jax.experimental.pallas import tpu as pltpu
from jax.experimental.pallas import tpu_sc as plsc

_N = 10000
_D = 128
_NP = 10240
_ROWS_PER_TILE = _NP // 16
_DINV_PER_WORKER = _NP // 32
_DEG_CHUNK = 512
_MSG_CHUNK = 256
_ROW_BLOCK = 1000

_mesh = plsc.VectorSubcoreMesh(core_axis_name="c", subcore_axis_name="s")


def _rsqrt_nr(x):
    xb = lax.bitcast_convert_type(x, jnp.int32)
    y = lax.bitcast_convert_type(jnp.int32(0x5F3759DF) - (xb >> 1), jnp.float32)
    for _ in range(3):
        y = y * (1.5 - 0.5 * x * y * y)
    return y


def _make_deg_dinv(epad):
    edges_per_tile = epad // 16
    n_chunks = edges_per_tile // _DEG_CHUNK
    assert n_chunks * _DEG_CHUNK == edges_per_tile

    @functools.partial(
        pl.kernel,
        mesh=_mesh,
        out_type=jax.ShapeDtypeStruct((_NP,), jnp.float32),
        scratch_types=[
            pltpu.VMEM_SHARED((_NP,), jnp.float32),
            pltpu.VMEM((_DEG_CHUNK,), jnp.int32),
            pltpu.VMEM((_DEG_CHUNK,), jnp.float32),
            pltpu.VMEM((_DINV_PER_WORKER,), jnp.float32),
        ],
    )
    def deg_dinv(dst_hbm, ew_hbm, z1_hbm, dinv_hbm, deg_sh, dst_v, ew_v, dv):
        cid = lax.axis_index("c")
        sid = lax.axis_index("s")
        pltpu.sync_copy(
            z1_hbm.at[pl.ds(sid * _ROWS_PER_TILE, _ROWS_PER_TILE)],
            deg_sh.at[pl.ds(sid * _ROWS_PER_TILE, _ROWS_PER_TILE)],
        )
        plsc.subcore_barrier()

        base = sid * edges_per_tile

        def body(i, carry):
            off = base + i * _DEG_CHUNK
            pltpu.sync_copy(dst_hbm.at[pl.ds(off, _DEG_CHUNK)], dst_v)
            pltpu.sync_copy(ew_hbm.at[pl.ds(off, _DEG_CHUNK)], ew_v)
            pltpu.sync_copy(ew_v, deg_sh.at[dst_v], add=True)
            return carry

        lax.fori_loop(0, n_chunks, body, 0)
        plsc.subcore_barrier()

        wid = cid * 16 + sid
        off = wid * _DINV_PER_WORKER
        pltpu.sync_copy(deg_sh.at[pl.ds(off, _DINV_PER_WORKER)], dv)
        for j in range(_DINV_PER_WORKER // 16):
            dv[pl.ds(j * 16, 16)] = _rsqrt_nr(dv[pl.ds(j * 16, 16)])
        pltpu.sync_copy(dv, dinv_hbm.at[pl.ds(off, _DINV_PER_WORKER)])

    return deg_dinv


def _make_messages(epad):
    edges_per_worker = epad // 32
    n_chunks = edges_per_worker // _MSG_CHUNK

    @functools.partial(
        pl.kernel,
        mesh=_mesh,
        out_type=jax.ShapeDtypeStruct((2, _NP, _D), jnp.float32),
        scratch_types=[
            pltpu.VMEM_SHARED((_NP, _D), jnp.float32),
            pltpu.VMEM((_MSG_CHUNK,), jnp.int32),
            pltpu.VMEM((_MSG_CHUNK,), jnp.int32),
            pltpu.VMEM((_MSG_CHUNK,), jnp.float32),
            pltpu.VMEM((_MSG_CHUNK,), jnp.float32),
            pltpu.VMEM((_MSG_CHUNK,), jnp.float32),
            pltpu.VMEM((_MSG_CHUNK, _D), jnp.float32),
            pltpu.SemaphoreType.DMA,
            pltpu.SemaphoreType.DMA,
            pltpu.SemaphoreType.DMA,
        ],
    )
    def messages(src_hbm, dst_hbm, ew_hbm, dinv_hbm, xw_hbm, z2_hbm, acc_hbm,
                 acc_sh, src_v, dst_v, ew_v, dsrc_v, ddst_v, rows_v,
                 sem, sem2, sem3):
        cid = lax.axis_index("c")
        sid = lax.axis_index("s")
        wid = cid * 16 + sid

        pltpu.sync_copy(
            z2_hbm.at[pl.ds(sid * _ROWS_PER_TILE, _ROWS_PER_TILE)],
            acc_sh.at[pl.ds(sid * _ROWS_PER_TILE, _ROWS_PER_TILE)],
        )
        plsc.subcore_barrier()

        ebase = wid * edges_per_worker

        def chunk(i, carry):
            off = ebase + i * _MSG_CHUNK
            pltpu.sync_copy(src_hbm.at[pl.ds(off, _MSG_CHUNK)], src_v)
            pltpu.sync_copy(dst_hbm.at[pl.ds(off, _MSG_CHUNK)], dst_v)
            pltpu.sync_copy(ew_hbm.at[pl.ds(off, _MSG_CHUNK)], ew_v)
            g1 = pltpu.async_copy(xw_hbm.at[src_v], rows_v, sem)
            g2 = pltpu.async_copy(dinv_hbm.at[src_v], dsrc_v, sem2)
            g3 = pltpu.async_copy(dinv_hbm.at[dst_v], ddst_v, sem3)
            g2.wait()
            g3.wait()
            g1.wait()

            def sbody(g, c):
                e16 = ew_v[pl.ds(g * 16, 16)]
                dsrc = dsrc_v[pl.ds(g * 16, 16)]
                ddst = ddst_v[pl.ds(g * 16, 16)]
                nv = dsrc * e16 * ddst
                for l in range(16):
                    e = g * 16 + l
                    nrm = nv[l]
                    for j in range(_D // 16):
                        rows_v[e, pl.ds(j * 16, 16)] = (
                            rows_v[e, pl.ds(j * 16, 16)] * nrm)
                return c

            lax.fori_loop(0, _MSG_CHUNK // 16, sbody, 0)

            pltpu.sync_copy(rows_v, acc_sh.at[dst_v], add=True)
            return carry

        lax.fori_loop(0, n_chunks, chunk, 0)
        plsc.subcore_barrier()

        pltpu.sync_copy(
            acc_sh.at[pl.ds(sid * _ROWS_PER_TILE, _ROWS_PER_TILE)],
            acc_hbm.at[cid, pl.ds(sid * _ROWS_PER_TILE, _ROWS_PER_TILE)],
        )

    return messages


def _matmul(x, w):
    def body(x_ref, w_ref, o_ref):
        o_ref[...] = lax.dot_general(
            x_ref[...], w_ref[...], (((1,), (1,)), ((), ())),
            preferred_element_type=jnp.float32)

    return pl.pallas_call(
        body,
        grid=(_N // _ROW_BLOCK,),
        in_specs=[
            pl.BlockSpec((_ROW_BLOCK, _D), lambda i: (i, 0)),
            pl.BlockSpec((_D, _D), lambda i: (0, 0)),
        ],
        out_specs=pl.BlockSpec((_ROW_BLOCK, _D), lambda i: (i, 0)),
        out_shape=jax.ShapeDtypeStruct((_N, _D), jnp.float32),
    )(x, w)


def _combine_stats(acc, b2):
    def body(acc_ref, b_ref, h_ref, st_ref):
        i = pl.program_id(0)
        blk = acc_ref[0] + acc_ref[1] + b_ref[...]
        h_ref[...] = blk
        s = jnp.sum(blk, axis=0, keepdims=True)
        ss = jnp.sum(blk * blk, axis=0, keepdims=True)
        st = jnp.concatenate([s, ss], axis=0)

        @pl.when(i == 0)
        def _():
            st_ref[...] = st

        @pl.when(i > 0)
        def _():
            st_ref[...] += st

    return pl.pallas_call(
        body,
        grid=(_N // _ROW_BLOCK,),
        in_specs=[
            pl.BlockSpec((2, _ROW_BLOCK, _D), lambda i: (0, i, 0)),
            pl.BlockSpec((1, _D), lambda i: (0, 0)),
        ],
        out_specs=[
            pl.BlockSpec((_ROW_BLOCK, _D), lambda i: (i, 0)),
            pl.BlockSpec((2, _D), lambda i: (0, 0)),
        ],
        out_shape=[
            jax.ShapeDtypeStruct((_N, _D), jnp.float32),
            jax.ShapeDtypeStruct((2, _D), jnp.float32),
        ],
    )(acc, b2)


def _bn_prelu(h, stats, g2, bt2, a2):
    inv_n = 1.0 / _N

    def body(h_ref, st_ref, g_ref, bt_ref, a_ref, o_ref):
        mean = st_ref[0:1, :] * inv_n
        var = st_ref[1:2, :] * inv_n - mean * mean
        rstd = lax.rsqrt(var + 1e-5)
        y = (h_ref[...] - mean) * rstd * g_ref[...] + bt_ref[...]
        o_ref[...] = jnp.maximum(y, 0.0) + a_ref[...] * jnp.minimum(y, 0.0)

    return pl.pallas_call(
        body,
        grid=(_N // _ROW_BLOCK,),
        in_specs=[
            pl.BlockSpec((_ROW_BLOCK, _D), lambda i: (i, 0)),
            pl.BlockSpec((2, _D), lambda i: (0, 0)),
            pl.BlockSpec((1, _D), lambda i: (0, 0)),
            pl.BlockSpec((1, _D), lambda i: (0, 0)),
            pl.BlockSpec((1, 1), lambda i: (0, 0)),
        ],
        out_specs=pl.BlockSpec((_ROW_BLOCK, _D), lambda i: (i, 0)),
        out_shape=jax.ShapeDtypeStruct((_N, _D), jnp.float32),
    )(h, stats, g2, bt2, a2)


def kernel(x, edge_index, edge_attr, W1, b1, gamma1, beta1, a1):
    n = x.shape[0]
    e = edge_attr.shape[0]
    assert n == _N and x.shape[1] == _D

    grain = 32 * _MSG_CHUNK
    epad = ((e + _NP + grain - 1) // grain) * grain
    npad = epad - e - _NP

    loop_idx = jnp.minimum(jnp.arange(_NP, dtype=jnp.int32), n - 1)
    self_ew = (jnp.arange(_NP) < n).astype(jnp.float32)
    pad_idx = jnp.full((npad,), n - 1, dtype=jnp.int32)
    src_all = jnp.concatenate([edge_index[0], loop_idx, pad_idx])
    dst_all = jnp.concatenate([edge_index[1], loop_idx, pad_idx])
    ew_all = jnp.concatenate(
        [edge_attr, self_ew, jnp.zeros((npad,), jnp.float32)])

    z1 = jnp.zeros((_NP,), jnp.float32)
    z2 = jnp.zeros((_NP, _D), jnp.float32)

    dinv = _make_deg_dinv(epad)(dst_all, ew_all, z1)
    xw = _matmul(x, W1)
    acc = _make_messages(epad)(src_all, dst_all, ew_all, dinv, xw, z2)
    h, stats = _combine_stats(acc, b1.reshape(1, _D))
    out = _bn_prelu(h, stats, gamma1.reshape(1, _D), beta1.reshape(1, _D),
                    a1.reshape(1, 1))
    return out


# speedup vs baseline: 12.1977x; 1.1304x over previous
import functools

import jax
import jax.numpy as jnp
from jax import lax
from jax.experimental import pallas as pl
from jax.experimental.pallas import tpu as pltpu
from jax.experimental.pallas import tpu_sc as plsc

_N = 10000
_D = 128
_NP = 10240
_ROWS_PER_TILE = _NP // 16
_DINV_PER_WORKER = _NP // 32
_MSG_CHUNK = 160
_DEG_CHUNK = 2 * _MSG_CHUNK
_ROW_BLOCK = 1000

_mesh = plsc.VectorSubcoreMesh(core_axis_name="c", subcore_axis_name="s")


def _rsqrt_nr(x):
    xb = lax.bitcast_convert_type(x, jnp.int32)
    y = lax.bitcast_convert_type(jnp.int32(0x5F3759DF) - (xb >> 1), jnp.float32)
    for _ in range(3):
        y = y * (1.5 - 0.5 * x * y * y)
    return y


def _make_deg_dinv(epad):
    edges_per_tile = epad // 16
    n_chunks = edges_per_tile // _DEG_CHUNK
    assert n_chunks * _DEG_CHUNK == edges_per_tile

    @functools.partial(
        pl.kernel,
        mesh=_mesh,
        out_type=jax.ShapeDtypeStruct((_NP,), jnp.float32),
        scratch_types=[
            pltpu.VMEM_SHARED((_NP,), jnp.float32),
            pltpu.VMEM((_DEG_CHUNK,), jnp.int32),
            pltpu.VMEM((_DEG_CHUNK,), jnp.float32),
            pltpu.VMEM((_DINV_PER_WORKER,), jnp.float32),
        ],
    )
    def deg_dinv(dst_hbm, ew_hbm, z1_hbm, dinv_hbm, deg_sh, dst_v, ew_v, dv):
        cid = lax.axis_index("c")
        sid = lax.axis_index("s")
        pltpu.sync_copy(
            z1_hbm.at[pl.ds(sid * _ROWS_PER_TILE, _ROWS_PER_TILE)],
            deg_sh.at[pl.ds(sid * _ROWS_PER_TILE, _ROWS_PER_TILE)],
        )
        plsc.subcore_barrier()

        base = sid * edges_per_tile

        def body(i, carry):
            off = base + i * _DEG_CHUNK
            pltpu.sync_copy(dst_hbm.at[pl.ds(off, _DEG_CHUNK)], dst_v)
            pltpu.sync_copy(ew_hbm.at[pl.ds(off, _DEG_CHUNK)], ew_v)
            pltpu.sync_copy(ew_v, deg_sh.at[dst_v], add=True)
            return carry

        lax.fori_loop(0, n_chunks, body, 0)
        plsc.subcore_barrier()

        wid = cid * 16 + sid
        off = wid * _DINV_PER_WORKER
        pltpu.sync_copy(deg_sh.at[pl.ds(off, _DINV_PER_WORKER)], dv)
        for j in range(_DINV_PER_WORKER // 16):
            dv[pl.ds(j * 16, 16)] = _rsqrt_nr(dv[pl.ds(j * 16, 16)])
        pltpu.sync_copy(dv, dinv_hbm.at[pl.ds(off, _DINV_PER_WORKER)])

    return deg_dinv


def _make_messages(epad):
    edges_per_worker = epad // 32
    n_chunks = edges_per_worker // _MSG_CHUNK
    assert n_chunks * _MSG_CHUNK == edges_per_worker and n_chunks % 2 == 0
    C = _MSG_CHUNK

    @functools.partial(
        pl.kernel,
        mesh=_mesh,
        out_type=jax.ShapeDtypeStruct((2, _NP, _D), jnp.float32),
        scratch_types=(
            [pltpu.VMEM_SHARED((_NP, _D), jnp.float32),
             pltpu.VMEM_SHARED((_NP,), jnp.float32)]
            + [pltpu.VMEM((C,), jnp.int32)] * 4
            + [pltpu.VMEM((C,), jnp.float32)] * 6
            + [pltpu.VMEM((C, _D), jnp.float32)] * 2
            + [pltpu.SemaphoreType.DMA] * 14
        ),
    )
    def messages(src_hbm, dst_hbm, ew_hbm, dinv_hbm, xw_hbm, z2_hbm, acc_hbm,
                 acc_sh, dinv_sh,
                 src0, dst0, src1, dst1, ew0, ds0, dd0, ew1, ds1, dd1,
                 rows0, rows1,
                 l0a, l0b, l0c, g0r, g0s, g0d, s0,
                 l1a, l1b, l1c, g1r, g1s, g1d, s1):
        cid = lax.axis_index("c")
        sid = lax.axis_index("s")
        wid = cid * 16 + sid

        slots = (
            (src0, dst0, ew0, ds0, dd0, rows0, l0a, l0b, l0c, g0r, g0s, g0d, s0),
            (src1, dst1, ew1, ds1, dd1, rows1, l1a, l1b, l1c, g1r, g1s, g1d, s1),
        )

        pltpu.sync_copy(
            z2_hbm.at[pl.ds(sid * _ROWS_PER_TILE, _ROWS_PER_TILE)],
            acc_sh.at[pl.ds(sid * _ROWS_PER_TILE, _ROWS_PER_TILE)],
        )
        pltpu.sync_copy(
            dinv_hbm.at[pl.ds(sid * (_NP // 16), _NP // 16)],
            dinv_sh.at[pl.ds(sid * (_NP // 16), _NP // 16)],
        )
        plsc.subcore_barrier()

        ebase = wid * edges_per_worker

        def issue_linear(c, sl):
            off = ebase + c * C
            pltpu.async_copy(src_hbm.at[pl.ds(off, C)], sl[0], sl[6])
            pltpu.async_copy(dst_hbm.at[pl.ds(off, C)], sl[1], sl[7])
            pltpu.async_copy(ew_hbm.at[pl.ds(off, C)], sl[2], sl[8])

        def wait_linear(sl):
            pltpu.make_async_copy(src_hbm.at[pl.ds(0, C)], sl[0], sl[6]).wait()
            pltpu.make_async_copy(dst_hbm.at[pl.ds(0, C)], sl[1], sl[7]).wait()
            pltpu.make_async_copy(ew_hbm.at[pl.ds(0, C)], sl[2], sl[8]).wait()

        def issue_gathers(sl):
            pltpu.async_copy(xw_hbm.at[sl[0]], sl[5], sl[9])
            pltpu.async_copy(dinv_sh.at[sl[0]], sl[3], sl[10])
            pltpu.async_copy(dinv_sh.at[sl[1]], sl[4], sl[11])

        def wait_gathers(sl):
            pltpu.make_async_copy(xw_hbm.at[sl[0]], sl[5], sl[9]).wait()
            pltpu.make_async_copy(dinv_sh.at[sl[0]], sl[3], sl[10]).wait()
            pltpu.make_async_copy(dinv_sh.at[sl[1]], sl[4], sl[11]).wait()

        def compute_scale(sl):
            ew_v, ds_v, dd_v, rows_v = sl[2], sl[3], sl[4], sl[5]

            def sbody(g, c):
                nv = (ds_v[pl.ds(g * 16, 16)] * ew_v[pl.ds(g * 16, 16)]
                      * dd_v[pl.ds(g * 16, 16)])
                for l in range(16):
                    e = g * 16 + l
                    nrm = nv[l]
                    for j in range(_D // 16):
                        rows_v[e, pl.ds(j * 16, 16)] = (
                            rows_v[e, pl.ds(j * 16, 16)] * nrm)
                return c

            lax.fori_loop(0, C // 16, sbody, 0)

        for b in (0, 1):
            issue_linear(b, slots[b])
            wait_linear(slots[b])
            issue_gathers(slots[b])

        def body(i, carry):
            for b in (0, 1):
                c = 2 * i + b
                sl = slots[b]
                wait_gathers(sl)
                compute_scale(sl)
                pltpu.async_copy(sl[5], acc_sh.at[sl[1]], sl[12], add=True)
                pltpu.make_async_copy(sl[5], acc_sh.at[sl[1]], sl[12]).wait()

                @pl.when(c + 2 < n_chunks)
                def _():
                    issue_linear(c + 2, sl)
                    wait_linear(sl)
                    issue_gathers(sl)

            return carry

        lax.fori_loop(0, n_chunks // 2, body, 0)
        plsc.subcore_barrier()

        pltpu.sync_copy(
            acc_sh.at[pl.ds(sid * _ROWS_PER_TILE, _ROWS_PER_TILE)],
            acc_hbm.at[cid, pl.ds(sid * _ROWS_PER_TILE, _ROWS_PER_TILE)],
        )

    return messages


def _matmul(x, w):
    def body(x_ref, w_ref, o_ref):
        o_ref[...] = lax.dot_general(
            x_ref[...], w_ref[...], (((1,), (1,)), ((), ())),
            preferred_element_type=jnp.float32)

    return pl.pallas_call(
        body,
        grid=(_N // _ROW_BLOCK,),
        in_specs=[
            pl.BlockSpec((_ROW_BLOCK, _D), lambda i: (i, 0)),
            pl.BlockSpec((_D, _D), lambda i: (0, 0)),
        ],
        out_specs=pl.BlockSpec((_ROW_BLOCK, _D), lambda i: (i, 0)),
        out_shape=jax.ShapeDtypeStruct((_N, _D), jnp.float32),
    )(x, w)


def _combine_stats(acc, b2):
    def body(acc_ref, b_ref, h_ref, st_ref):
        i = pl.program_id(0)
        blk = acc_ref[0] + acc_ref[1] + b_ref[...]
        h_ref[...] = blk
        s = jnp.sum(blk, axis=0, keepdims=True)
        ss = jnp.sum(blk * blk, axis=0, keepdims=True)
        st = jnp.concatenate([s, ss], axis=0)

        @pl.when(i == 0)
        def _():
            st_ref[...] = st

        @pl.when(i > 0)
        def _():
            st_ref[...] += st

    return pl.pallas_call(
        body,
        grid=(_N // _ROW_BLOCK,),
        in_specs=[
            pl.BlockSpec((2, _ROW_BLOCK, _D), lambda i: (0, i, 0)),
            pl.BlockSpec((1, _D), lambda i: (0, 0)),
        ],
        out_specs=[
            pl.BlockSpec((_ROW_BLOCK, _D), lambda i: (i, 0)),
            pl.BlockSpec((2, _D), lambda i: (0, 0)),
        ],
        out_shape=[
            jax.ShapeDtypeStruct((_N, _D), jnp.float32),
            jax.ShapeDtypeStruct((2, _D), jnp.float32),
        ],
    )(acc, b2)


def _bn_prelu(h, stats, g2, bt2, a2):
    inv_n = 1.0 / _N

    def body(h_ref, st_ref, g_ref, bt_ref, a_ref, o_ref):
        mean = st_ref[0:1, :] * inv_n
        var = st_ref[1:2, :] * inv_n - mean * mean
        rstd = lax.rsqrt(var + 1e-5)
        y = (h_ref[...] - mean) * rstd * g_ref[...] + bt_ref[...]
        o_ref[...] = jnp.maximum(y, 0.0) + a_ref[...] * jnp.minimum(y, 0.0)

    return pl.pallas_call(
        body,
        grid=(_N // _ROW_BLOCK,),
        in_specs=[
            pl.BlockSpec((_ROW_BLOCK, _D), lambda i: (i, 0)),
            pl.BlockSpec((2, _D), lambda i: (0, 0)),
            pl.BlockSpec((1, _D), lambda i: (0, 0)),
            pl.BlockSpec((1, _D), lambda i: (0, 0)),
            pl.BlockSpec((1, 1), lambda i: (0, 0)),
        ],
        out_specs=pl.BlockSpec((_ROW_BLOCK, _D), lambda i: (i, 0)),
        out_shape=jax.ShapeDtypeStruct((_N, _D), jnp.float32),
    )(h, stats, g2, bt2, a2)


def kernel(x, edge_index, edge_attr, W1, b1, gamma1, beta1, a1):
    n = x.shape[0]
    e = edge_attr.shape[0]
    assert n == _N and x.shape[1] == _D

    grain = 64 * _MSG_CHUNK
    epad = ((e + _NP + grain - 1) // grain) * grain
    npad = epad - e - _NP

    loop_idx = jnp.minimum(jnp.arange(_NP, dtype=jnp.int32), n - 1)
    self_ew = (jnp.arange(_NP) < n).astype(jnp.float32)
    pad_idx = jnp.full((npad,), n - 1, dtype=jnp.int32)
    src_all = jnp.concatenate([edge_index[0], loop_idx, pad_idx])
    dst_all = jnp.concatenate([edge_index[1], loop_idx, pad_idx])
    ew_all = jnp.concatenate(
        [edge_attr, self_ew, jnp.zeros((npad,), jnp.float32)])

    z1 = jnp.zeros((_NP,), jnp.float32)
    z2 = jnp.zeros((_NP, _D), jnp.float32)

    dinv = _make_deg_dinv(epad)(dst_all, ew_all, z1)
    xw = _matmul(x, W1)
    acc = _make_messages(epad)(src_all, dst_all, ew_all, dinv, xw, z2)
    h, stats = _combine_stats(acc, b1.reshape(1, _D))
    out = _bn_prelu(h, stats, gamma1.reshape(1, _D), beta1.reshape(1, _D),
                    a1.reshape(1, 1))
    return out


# speedup vs baseline: 29.9486x; 2.4553x over previous
import functools

import jax
import jax.numpy as jnp
from jax import lax
from jax.experimental import pallas as pl
from jax.experimental.pallas import tpu as pltpu
from jax.experimental.pallas import tpu_sc as plsc

_N = 10000
_D = 128
_NP = 10240
_ROWS_PER_TILE = _NP // 16
_DINV_PER_WORKER = _NP // 32
_MSG_CHUNK = 160
_ROW_BLOCK = 1000

_mesh = plsc.VectorSubcoreMesh(core_axis_name="c", subcore_axis_name="s")


def _rsqrt_nr(x):
    xb = lax.bitcast_convert_type(x, jnp.int32)
    y = lax.bitcast_convert_type(jnp.int32(0x5F3759DF) - (xb >> 1), jnp.float32)
    for _ in range(3):
        y = y * (1.5 - 0.5 * x * y * y)
    return y


def _make_deg_dinv(epad):
    edges_per_tile = epad // 16
    n_chunks = 16
    deg_chunk = edges_per_tile // n_chunks
    assert n_chunks * deg_chunk == edges_per_tile and deg_chunk % 8 == 0

    @functools.partial(
        pl.kernel,
        mesh=_mesh,
        out_type=jax.ShapeDtypeStruct((_NP,), jnp.float32),
        scratch_types=[
            pltpu.VMEM_SHARED((_NP,), jnp.float32),
            pltpu.VMEM((deg_chunk,), jnp.int32),
            pltpu.VMEM((deg_chunk,), jnp.float32),
            pltpu.VMEM((_DINV_PER_WORKER,), jnp.float32),
        ],
    )
    def deg_dinv(dst_hbm, ew_hbm, z1_hbm, dinv_hbm, deg_sh, dst_v, ew_v, dv):
        cid = lax.axis_index("c")
        sid = lax.axis_index("s")
        pltpu.sync_copy(
            z1_hbm.at[pl.ds(sid * _ROWS_PER_TILE, _ROWS_PER_TILE)],
            deg_sh.at[pl.ds(sid * _ROWS_PER_TILE, _ROWS_PER_TILE)],
        )
        plsc.subcore_barrier()

        base = sid * edges_per_tile

        def body(i, carry):
            off = base + i * deg_chunk
            pltpu.sync_copy(dst_hbm.at[pl.ds(off, deg_chunk)], dst_v)
            pltpu.sync_copy(ew_hbm.at[pl.ds(off, deg_chunk)], ew_v)
            pltpu.sync_copy(ew_v, deg_sh.at[dst_v], add=True)
            return carry

        lax.fori_loop(0, n_chunks, body, 0)
        plsc.subcore_barrier()

        wid = cid * 16 + sid
        off = wid * _DINV_PER_WORKER
        pltpu.sync_copy(deg_sh.at[pl.ds(off, _DINV_PER_WORKER)], dv)
        for j in range(_DINV_PER_WORKER // 16):
            dv[pl.ds(j * 16, 16)] = _rsqrt_nr(dv[pl.ds(j * 16, 16)])
        pltpu.sync_copy(dv, dinv_hbm.at[pl.ds(off, _DINV_PER_WORKER)])

    return deg_dinv


def _make_messages(epad):
    edges_per_worker = epad // 32
    n_chunks = edges_per_worker // _MSG_CHUNK
    assert n_chunks * _MSG_CHUNK == edges_per_worker and n_chunks % 2 == 0
    C = _MSG_CHUNK

    @functools.partial(
        pl.kernel,
        mesh=_mesh,
        out_type=jax.ShapeDtypeStruct((2, _NP, _D), jnp.float32),
        scratch_types=(
            [pltpu.VMEM_SHARED((_NP, _D), jnp.float32),
             pltpu.VMEM_SHARED((_NP,), jnp.float32)]
            + [pltpu.VMEM((C,), jnp.int32)] * 4
            + [pltpu.VMEM((C,), jnp.float32)] * 6
            + [pltpu.VMEM((C, _D), jnp.float32)] * 2
            + [pltpu.SemaphoreType.DMA] * 14
        ),
    )
    def messages(src_hbm, dst_hbm, ew_hbm, dinv_hbm, xw_hbm, z2_hbm, acc_hbm,
                 acc_sh, dinv_sh,
                 src0, dst0, src1, dst1, ew0, ds0, dd0, ew1, ds1, dd1,
                 rows0, rows1,
                 l0a, l0b, l0c, g0r, g0s, g0d, s0,
                 l1a, l1b, l1c, g1r, g1s, g1d, s1):
        cid = lax.axis_index("c")
        sid = lax.axis_index("s")
        wid = cid * 16 + sid

        slots = (
            (src0, dst0, ew0, ds0, dd0, rows0, l0a, l0b, l0c, g0r, g0s, g0d, s0),
            (src1, dst1, ew1, ds1, dd1, rows1, l1a, l1b, l1c, g1r, g1s, g1d, s1),
        )

        pltpu.sync_copy(
            z2_hbm.at[pl.ds(sid * _ROWS_PER_TILE, _ROWS_PER_TILE)],
            acc_sh.at[pl.ds(sid * _ROWS_PER_TILE, _ROWS_PER_TILE)],
        )
        pltpu.sync_copy(
            dinv_hbm.at[pl.ds(sid * (_NP // 16), _NP // 16)],
            dinv_sh.at[pl.ds(sid * (_NP // 16), _NP // 16)],
        )
        plsc.subcore_barrier()

        ebase = wid * edges_per_worker

        def issue_linear(c, sl):
            off = ebase + c * C
            pltpu.async_copy(src_hbm.at[pl.ds(off, C)], sl[0], sl[6])
            pltpu.async_copy(dst_hbm.at[pl.ds(off, C)], sl[1], sl[7])
            pltpu.async_copy(ew_hbm.at[pl.ds(off, C)], sl[2], sl[8])

        def wait_linear(sl):
            pltpu.make_async_copy(src_hbm.at[pl.ds(0, C)], sl[0], sl[6]).wait()
            pltpu.make_async_copy(dst_hbm.at[pl.ds(0, C)], sl[1], sl[7]).wait()
            pltpu.make_async_copy(ew_hbm.at[pl.ds(0, C)], sl[2], sl[8]).wait()

        def issue_gathers(sl):
            pltpu.async_copy(xw_hbm.at[sl[0]], sl[5], sl[9])
            pltpu.async_copy(dinv_sh.at[sl[0]], sl[3], sl[10])
            pltpu.async_copy(dinv_sh.at[sl[1]], sl[4], sl[11])

        def wait_gathers(sl):
            pltpu.make_async_copy(xw_hbm.at[sl[0]], sl[5], sl[9]).wait()
            pltpu.make_async_copy(dinv_sh.at[sl[0]], sl[3], sl[10]).wait()
            pltpu.make_async_copy(dinv_sh.at[sl[1]], sl[4], sl[11]).wait()

        def compute_scale(sl):
            ew_v, ds_v, dd_v, rows_v = sl[2], sl[3], sl[4], sl[5]

            def sbody(g, c):
                nv = (ds_v[pl.ds(g * 16, 16)] * ew_v[pl.ds(g * 16, 16)]
                      * dd_v[pl.ds(g * 16, 16)])
                for l in range(16):
                    e = g * 16 + l
                    nrm = nv[l]
                    for j in range(_D // 16):
                        rows_v[e, pl.ds(j * 16, 16)] = (
                            rows_v[e, pl.ds(j * 16, 16)] * nrm)
                return c

            lax.fori_loop(0, C // 16, sbody, 0)

        for b in (0, 1):
            issue_linear(b, slots[b])
            wait_linear(slots[b])
            issue_gathers(slots[b])

        def body(i, carry):
            for b in (0, 1):
                c = 2 * i + b
                sl = slots[b]
                wait_gathers(sl)
                compute_scale(sl)
                pltpu.async_copy(sl[5], acc_sh.at[sl[1]], sl[12], add=True)
                pltpu.make_async_copy(sl[5], acc_sh.at[sl[1]], sl[12]).wait()

                @pl.when(c + 2 < n_chunks)
                def _():
                    issue_linear(c + 2, sl)
                    wait_linear(sl)
                    issue_gathers(sl)

            return carry

        lax.fori_loop(0, n_chunks // 2, body, 0)
        plsc.subcore_barrier()

        pltpu.sync_copy(
            acc_sh.at[pl.ds(sid * _ROWS_PER_TILE, _ROWS_PER_TILE)],
            acc_hbm.at[cid, pl.ds(sid * _ROWS_PER_TILE, _ROWS_PER_TILE)],
        )

    return messages


def _matmul(x, w):
    def body(x_ref, w_ref, o_ref):
        o_ref[...] = lax.dot_general(
            x_ref[...], w_ref[...], (((1,), (1,)), ((), ())),
            preferred_element_type=jnp.float32)

    return pl.pallas_call(
        body,
        grid=(_N // _ROW_BLOCK,),
        in_specs=[
            pl.BlockSpec((_ROW_BLOCK, _D), lambda i: (i, 0)),
            pl.BlockSpec((_D, _D), lambda i: (0, 0)),
        ],
        out_specs=pl.BlockSpec((_ROW_BLOCK, _D), lambda i: (i, 0)),
        out_shape=jax.ShapeDtypeStruct((_N, _D), jnp.float32),
    )(x, w)


def _combine_stats(acc, b2):
    def body(acc_ref, b_ref, h_ref, st_ref):
        i = pl.program_id(0)
        blk = acc_ref[0] + acc_ref[1] + b_ref[...]
        h_ref[...] = blk
        s = jnp.sum(blk, axis=0, keepdims=True)
        ss = jnp.sum(blk * blk, axis=0, keepdims=True)
        st = jnp.concatenate([s, ss], axis=0)

        @pl.when(i == 0)
        def _():
            st_ref[...] = st

        @pl.when(i > 0)
        def _():
            st_ref[...] += st

    return pl.pallas_call(
        body,
        grid=(_N // _ROW_BLOCK,),
        in_specs=[
            pl.BlockSpec((2, _ROW_BLOCK, _D), lambda i: (0, i, 0)),
            pl.BlockSpec((1, _D), lambda i: (0, 0)),
        ],
        out_specs=[
            pl.BlockSpec((_ROW_BLOCK, _D), lambda i: (i, 0)),
            pl.BlockSpec((2, _D), lambda i: (0, 0)),
        ],
        out_shape=[
            jax.ShapeDtypeStruct((_N, _D), jnp.float32),
            jax.ShapeDtypeStruct((2, _D), jnp.float32),
        ],
    )(acc, b2)


def _bn_prelu(h, stats, g2, bt2, a2):
    inv_n = 1.0 / _N

    def body(h_ref, st_ref, g_ref, bt_ref, a_ref, o_ref):
        mean = st_ref[0:1, :] * inv_n
        var = st_ref[1:2, :] * inv_n - mean * mean
        rstd = lax.rsqrt(var + 1e-5)
        y = (h_ref[...] - mean) * rstd * g_ref[...] + bt_ref[...]
        o_ref[...] = jnp.maximum(y, 0.0) + a_ref[...] * jnp.minimum(y, 0.0)

    return pl.pallas_call(
        body,
        grid=(_N // _ROW_BLOCK,),
        in_specs=[
            pl.BlockSpec((_ROW_BLOCK, _D), lambda i: (i, 0)),
            pl.BlockSpec((2, _D), lambda i: (0, 0)),
            pl.BlockSpec((1, _D), lambda i: (0, 0)),
            pl.BlockSpec((1, _D), lambda i: (0, 0)),
            pl.BlockSpec((1, 1), lambda i: (0, 0)),
        ],
        out_specs=pl.BlockSpec((_ROW_BLOCK, _D), lambda i: (i, 0)),
        out_shape=jax.ShapeDtypeStruct((_N, _D), jnp.float32),
    )(h, stats, g2, bt2, a2)


def kernel(x, edge_index, edge_attr, W1, b1, gamma1, beta1, a1):
    n = x.shape[0]
    e = edge_attr.shape[0]
    assert n == _N and x.shape[1] == _D

    grain = 64 * _MSG_CHUNK
    epad = ((e + _NP + grain - 1) // grain) * grain
    npad = epad - e - _NP

    loop_idx = jnp.minimum(jnp.arange(_NP, dtype=jnp.int32), n - 1)
    self_ew = (jnp.arange(_NP) < n).astype(jnp.float32)
    pad_idx = (jnp.arange(npad, dtype=jnp.int32) * 37) % n
    src_all = jnp.concatenate([edge_index[0], loop_idx, pad_idx])
    dst_all = jnp.concatenate([edge_index[1], loop_idx, pad_idx])
    ew_all = jnp.concatenate(
        [edge_attr, self_ew, jnp.zeros((npad,), jnp.float32)])

    z1 = jnp.zeros((_NP,), jnp.float32)
    z2 = jnp.zeros((_NP, _D), jnp.float32)

    dinv = _make_deg_dinv(epad)(dst_all, ew_all, z1)
    xw = _matmul(x, W1)
    acc = _make_messages(epad)(src_all, dst_all, ew_all, dinv, xw, z2)
    h, stats = _combine_stats(acc, b1.reshape(1, _D))
    out = _bn_prelu(h, stats, gamma1.reshape(1, _D), beta1.reshape(1, _D),
                    a1.reshape(1, 1))
    return out


# speedup vs baseline: 34.9688x; 1.1676x over previous
import functools

import jax
import jax.numpy as jnp
from jax import lax
from jax.experimental import pallas as pl
from jax.experimental.pallas import tpu as pltpu
from jax.experimental.pallas import tpu_sc as plsc

_N = 10000
_D = 128
_NP = 10240
_ROWS_PER_TILE = _NP // 16
_DINV_PER_WORKER = _NP // 32
_MSG_CHUNK = 112
_ROW_BLOCK = 1000

_mesh = plsc.VectorSubcoreMesh(core_axis_name="c", subcore_axis_name="s")


def _rsqrt_nr(x):
    xb = lax.bitcast_convert_type(x, jnp.int32)
    y = lax.bitcast_convert_type(jnp.int32(0x5F3759DF) - (xb >> 1), jnp.float32)
    for _ in range(3):
        y = y * (1.5 - 0.5 * x * y * y)
    return y


def _pick_deg_chunk(edges_per_tile):
    for c in range(2048, 0, -8):
        if edges_per_tile % c == 0:
            return c
    raise ValueError(edges_per_tile)


def _make_deg_dinv(epad):
    edges_per_tile = epad // 16
    deg_chunk = _pick_deg_chunk(edges_per_tile)
    n_chunks = edges_per_tile // deg_chunk

    @functools.partial(
        pl.kernel,
        mesh=_mesh,
        out_type=jax.ShapeDtypeStruct((_NP,), jnp.float32),
        scratch_types=[
            pltpu.VMEM_SHARED((_NP,), jnp.float32),
            pltpu.VMEM((deg_chunk,), jnp.int32),
            pltpu.VMEM((deg_chunk,), jnp.float32),
            pltpu.VMEM((_DINV_PER_WORKER,), jnp.float32),
        ],
    )
    def deg_dinv(dst_hbm, ew_hbm, z1_hbm, dinv_hbm, deg_sh, dst_v, ew_v, dv):
        cid = lax.axis_index("c")
        sid = lax.axis_index("s")
        pltpu.sync_copy(
            z1_hbm.at[pl.ds(sid * _ROWS_PER_TILE, _ROWS_PER_TILE)],
            deg_sh.at[pl.ds(sid * _ROWS_PER_TILE, _ROWS_PER_TILE)],
        )
        plsc.subcore_barrier()

        base = sid * edges_per_tile

        def body(i, carry):
            off = base + i * deg_chunk
            pltpu.sync_copy(dst_hbm.at[pl.ds(off, deg_chunk)], dst_v)
            pltpu.sync_copy(ew_hbm.at[pl.ds(off, deg_chunk)], ew_v)
            pltpu.sync_copy(ew_v, deg_sh.at[dst_v], add=True)
            return carry

        lax.fori_loop(0, n_chunks, body, 0)
        plsc.subcore_barrier()

        wid = cid * 16 + sid
        off = wid * _DINV_PER_WORKER
        pltpu.sync_copy(deg_sh.at[pl.ds(off, _DINV_PER_WORKER)], dv)
        for j in range(_DINV_PER_WORKER // 16):
            dv[pl.ds(j * 16, 16)] = _rsqrt_nr(dv[pl.ds(j * 16, 16)])
        pltpu.sync_copy(dv, dinv_hbm.at[pl.ds(off, _DINV_PER_WORKER)])

    return deg_dinv


def _make_messages(epad):
    edges_per_worker = epad // 32
    n_chunks = edges_per_worker // _MSG_CHUNK
    assert n_chunks * _MSG_CHUNK == edges_per_worker and n_chunks % 6 == 0
    C = _MSG_CHUNK
    NI = 6
    NR = 3

    @functools.partial(
        pl.kernel,
        mesh=_mesh,
        out_type=jax.ShapeDtypeStruct((2, _NP, _D), jnp.float32),
        scratch_types=(
            [pltpu.VMEM_SHARED((_NP, _D), jnp.float32),
             pltpu.VMEM_SHARED((_NP,), jnp.float32)]
            + [pltpu.VMEM((C,), jnp.int32)] * (2 * NI)
            + [pltpu.VMEM((C,), jnp.float32)] * NI
            + [pltpu.VMEM((C,), jnp.float32)] * (2 * NR)
            + [pltpu.VMEM((C, _D), jnp.float32)] * NR
            + [pltpu.SemaphoreType.DMA] * (3 * NI + 3 * NR + NR)
        ),
    )
    def messages(src_hbm, dst_hbm, ew_hbm, dinv_hbm, xw_hbm, z2_hbm, acc_hbm,
                 *scr):
        acc_sh, dinv_sh = scr[0], scr[1]
        p = 2
        srcs = scr[p:p + NI]; p += NI
        dsts = scr[p:p + NI]; p += NI
        ews = scr[p:p + NI]; p += NI
        dss = scr[p:p + NR]; p += NR
        dds = scr[p:p + NR]; p += NR
        rows = scr[p:p + NR]; p += NR
        semL = scr[p:p + 3 * NI]; p += 3 * NI
        semG = scr[p:p + 3 * NR]; p += 3 * NR
        semS = scr[p:p + NR]; p += NR

        cid = lax.axis_index("c")
        sid = lax.axis_index("s")
        wid = cid * 16 + sid

        pltpu.sync_copy(
            z2_hbm.at[pl.ds(sid * _ROWS_PER_TILE, _ROWS_PER_TILE)],
            acc_sh.at[pl.ds(sid * _ROWS_PER_TILE, _ROWS_PER_TILE)],
        )
        pltpu.sync_copy(
            dinv_hbm.at[pl.ds(sid * (_NP // 16), _NP // 16)],
            dinv_sh.at[pl.ds(sid * (_NP // 16), _NP // 16)],
        )
        plsc.subcore_barrier()

        ebase = wid * edges_per_worker

        def issue_L(c, k):
            off = ebase + c * C
            pltpu.async_copy(src_hbm.at[pl.ds(off, C)], srcs[k], semL[3 * k])
            pltpu.async_copy(dst_hbm.at[pl.ds(off, C)], dsts[k],
                             semL[3 * k + 1])
            pltpu.async_copy(ew_hbm.at[pl.ds(off, C)], ews[k], semL[3 * k + 2])

        def wait_L(k):
            z = src_hbm.at[pl.ds(0, C)]
            pltpu.make_async_copy(z, srcs[k], semL[3 * k]).wait()
            pltpu.make_async_copy(z, dsts[k], semL[3 * k + 1]).wait()
            pltpu.make_async_copy(ew_hbm.at[pl.ds(0, C)], ews[k],
                                  semL[3 * k + 2]).wait()

        def issue_G(k, r):
            pltpu.async_copy(xw_hbm.at[srcs[k]], rows[r], semG[3 * r])
            pltpu.async_copy(dinv_sh.at[srcs[k]], dss[r], semG[3 * r + 1])
            pltpu.async_copy(dinv_sh.at[dsts[k]], dds[r], semG[3 * r + 2])

        def wait_G(k, r):
            pltpu.make_async_copy(xw_hbm.at[srcs[k]], rows[r],
                                  semG[3 * r]).wait()
            pltpu.make_async_copy(dinv_sh.at[srcs[k]], dss[r],
                                  semG[3 * r + 1]).wait()
            pltpu.make_async_copy(dinv_sh.at[dsts[k]], dds[r],
                                  semG[3 * r + 2]).wait()

        def issue_S(k, r):
            pltpu.async_copy(rows[r], acc_sh.at[dsts[k]], semS[r], add=True)

        def wait_S(k, r):
            pltpu.make_async_copy(rows[r], acc_sh.at[dsts[k]], semS[r]).wait()

        def compute_scale(k, r):
            ew_v, ds_v, dd_v, rows_v = ews[k], dss[r], dds[r], rows[r]

            def sbody(g, c):
                nv = (ds_v[pl.ds(g * 16, 16)] * ew_v[pl.ds(g * 16, 16)]
                      * dd_v[pl.ds(g * 16, 16)])
                for l in range(16):
                    e = g * 16 + l
                    nrm = nv[l]
                    for j in range(_D // 16):
                        rows_v[e, pl.ds(j * 16, 16)] = (
                            rows_v[e, pl.ds(j * 16, 16)] * nrm)
                return c

            lax.fori_loop(0, C // 16, sbody, 0)

        for c0 in range(4):
            issue_L(c0, c0)
        for c0 in range(2):
            wait_L(c0)
            issue_G(c0, c0)

        def body(i, carry):
            for k in range(NI):
                c = 6 * i + k
                r = k % NR
                wait_G(k, r)
                compute_scale(k, r)
                issue_S(k, r)

                kp, rp = (k - 1) % NI, (k - 1) % NR
                if k == 0:
                    @pl.when(i >= 1)
                    def _():
                        wait_S(kp, rp)
                else:
                    wait_S(kp, rp)

                @pl.when(c + 4 < n_chunks)
                def _():
                    issue_L(c + 4, (k + 4) % NI)

                @pl.when(c + 2 < n_chunks)
                def _():
                    wait_L((k + 2) % NI)
                    issue_G((k + 2) % NI, (k + 2) % NR)

            return carry

        lax.fori_loop(0, n_chunks // 6, body, 0)
        wait_S((n_chunks - 1) % NI, (n_chunks - 1) % NR)
        plsc.subcore_barrier()

        pltpu.sync_copy(
            acc_sh.at[pl.ds(sid * _ROWS_PER_TILE, _ROWS_PER_TILE)],
            acc_hbm.at[cid, pl.ds(sid * _ROWS_PER_TILE, _ROWS_PER_TILE)],
        )

    return messages


def _matmul(x, w):
    def body(x_ref, w_ref, o_ref):
        o_ref[...] = lax.dot_general(
            x_ref[...], w_ref[...], (((1,), (1,)), ((), ())),
            preferred_element_type=jnp.float32)

    return pl.pallas_call(
        body,
        grid=(_N // _ROW_BLOCK,),
        in_specs=[
            pl.BlockSpec((_ROW_BLOCK, _D), lambda i: (i, 0)),
            pl.BlockSpec((_D, _D), lambda i: (0, 0)),
        ],
        out_specs=pl.BlockSpec((_ROW_BLOCK, _D), lambda i: (i, 0)),
        out_shape=jax.ShapeDtypeStruct((_N, _D), jnp.float32),
    )(x, w)


def _combine_stats(acc, b2):
    def body(acc_ref, b_ref, h_ref, st_ref):
        i = pl.program_id(0)
        blk = acc_ref[0] + acc_ref[1] + b_ref[...]
        h_ref[...] = blk
        s = jnp.sum(blk, axis=0, keepdims=True)
        ss = jnp.sum(blk * blk, axis=0, keepdims=True)
        st = jnp.concatenate([s, ss], axis=0)

        @pl.when(i == 0)
        def _():
            st_ref[...] = st

        @pl.when(i > 0)
        def _():
            st_ref[...] += st

    return pl.pallas_call(
        body,
        grid=(_N // _ROW_BLOCK,),
        in_specs=[
            pl.BlockSpec((2, _ROW_BLOCK, _D), lambda i: (0, i, 0)),
            pl.BlockSpec((1, _D), lambda i: (0, 0)),
        ],
        out_specs=[
            pl.BlockSpec((_ROW_BLOCK, _D), lambda i: (i, 0)),
            pl.BlockSpec((2, _D), lambda i: (0, 0)),
        ],
        out_shape=[
            jax.ShapeDtypeStruct((_N, _D), jnp.float32),
            jax.ShapeDtypeStruct((2, _D), jnp.float32),
        ],
    )(acc, b2)


def _bn_prelu(h, stats, g2, bt2, a2):
    inv_n = 1.0 / _N

    def body(h_ref, st_ref, g_ref, bt_ref, a_ref, o_ref):
        mean = st_ref[0:1, :] * inv_n
        var = st_ref[1:2, :] * inv_n - mean * mean
        rstd = lax.rsqrt(var + 1e-5)
        y = (h_ref[...] - mean) * rstd * g_ref[...] + bt_ref[...]
        o_ref[...] = jnp.maximum(y, 0.0) + a_ref[...] * jnp.minimum(y, 0.0)

    return pl.pallas_call(
        body,
        grid=(_N // _ROW_BLOCK,),
        in_specs=[
            pl.BlockSpec((_ROW_BLOCK, _D), lambda i: (i, 0)),
            pl.BlockSpec((2, _D), lambda i: (0, 0)),
            pl.BlockSpec((1, _D), lambda i: (0, 0)),
            pl.BlockSpec((1, _D), lambda i: (0, 0)),
            pl.BlockSpec((1, 1), lambda i: (0, 0)),
        ],
        out_specs=pl.BlockSpec((_ROW_BLOCK, _D), lambda i: (i, 0)),
        out_shape=jax.ShapeDtypeStruct((_N, _D), jnp.float32),
    )(h, stats, g2, bt2, a2)


def kernel(x, edge_index, edge_attr, W1, b1, gamma1, beta1, a1):
    n = x.shape[0]
    e = edge_attr.shape[0]
    assert n == _N and x.shape[1] == _D

    grain = 6 * 32 * _MSG_CHUNK
    epad = ((e + _NP + grain - 1) // grain) * grain
    npad = epad - e - _NP

    loop_idx = jnp.minimum(jnp.arange(_NP, dtype=jnp.int32), n - 1)
    self_ew = (jnp.arange(_NP) < n).astype(jnp.float32)
    pad_idx = (jnp.arange(npad, dtype=jnp.int32) * 37) % n
    src_all = jnp.concatenate([edge_index[0], loop_idx, pad_idx])
    dst_all = jnp.concatenate([edge_index[1], loop_idx, pad_idx])
    ew_all = jnp.concatenate(
        [edge_attr, self_ew, jnp.zeros((npad,), jnp.float32)])

    z1 = jnp.zeros((_NP,), jnp.float32)
    z2 = jnp.zeros((_NP, _D), jnp.float32)

    dinv = _make_deg_dinv(epad)(dst_all, ew_all, z1)
    xw = _matmul(x, W1)
    acc = _make_messages(epad)(src_all, dst_all, ew_all, dinv, xw, z2)
    h, stats = _combine_stats(acc, b1.reshape(1, _D))
    out = _bn_prelu(h, stats, gamma1.reshape(1, _D), beta1.reshape(1, _D),
                    a1.reshape(1, 1))
    return out
